# Initial kernel scaffold; baseline (speedup 1.0000x reference)
import jax
import jax.numpy as jnp
from jax.experimental import pallas as pl


def kernel(fragments, alphas, ptclds, im):
    raise NotImplementedError("write your pallas kernel here")


# speedup vs baseline: 85.1514x; 85.1514x over previous
import functools

import jax
import jax.numpy as jnp
from jax import lax
from jax.experimental import pallas as pl
from jax.experimental.pallas import tpu as pltpu
from jax.experimental.pallas import tpu_sc as plsc

NC, NS = 2, 16
NW = NC * NS
PAIRS = NW // 2
LANES = 16


def _sc_composite(tbl, frags, alphas, im4):
    B, K, HW = frags.shape
    P = tbl.shape[1]
    ppb = HW // PAIRS
    n = min(512, ppb)
    nchunk = ppb // n
    ngroups = n // LANES
    total_chunks = B * nchunk

    mesh = plsc.VectorSubcoreMesh(core_axis_name="c", subcore_axis_name="s")

    @functools.partial(
        pl.kernel,
        out_type=jax.ShapeDtypeStruct((B, 4, HW), jnp.float32),
        mesh=mesh,
        compiler_params=pltpu.CompilerParams(needs_layout_passes=False),
        scratch_types=[
            pltpu.VMEM((P,), jnp.int32),
            pltpu.VMEM((K, n), jnp.int32),
            pltpu.VMEM((K, n), jnp.float32),
            pltpu.VMEM((2, n), jnp.float32),
            pltpu.VMEM((2, n), jnp.float32),
        ],
    )
    def body(tbl_hbm, frag_hbm, alpha_hbm, im_hbm, out_hbm,
             tbl_v, frag_v, alpha_v, im_v, acc_v):
        wid = lax.axis_index("s") * NC + lax.axis_index("c")
        pair = wid // 2
        member = wid % 2
        ch0 = member * 2
        keep_hi = (ch0 + 1) < 3
        pltpu.sync_copy(tbl_hbm.at[member], tbl_v)

        def chunk_body(i, carry):
            b = i // nchunk
            off = pair * ppb + (i % nchunk) * n
            pltpu.sync_copy(frag_hbm.at[b, :, pl.ds(off, n)], frag_v)
            pltpu.sync_copy(alpha_hbm.at[b, :, pl.ds(off, n)], alpha_v)
            pltpu.sync_copy(im_hbm.at[pl.ds(ch0, 2), pl.ds(off, n)], im_v)

            def group_body(g, c2):
                s = pl.ds(g * LANES, LANES)
                t = jnp.ones((LANES,), jnp.float32)
                acc0 = jnp.zeros((LANES,), jnp.float32)
                acc1 = jnp.zeros((LANES,), jnp.float32)
                for kk in range(K):
                    idx = frag_v[kk, s]
                    al = alpha_v[kk, s]
                    m = idx >= 0
                    a = jnp.where(m, al, 0.0)
                    w = a * t
                    t = t * (1.0 - a)
                    safe = jnp.where(m, idx, 0)
                    gw = plsc.load_gather(tbl_v, [safe])
                    clo = plsc.bitcast(gw << 16, jnp.float32)
                    chi = plsc.bitcast(gw & jnp.int32(-65536), jnp.float32)
                    acc0 = acc0 + w * clo
                    acc1 = acc1 + w * chi
                bg = frag_v[0, s] < 0
                acc0 = jnp.where(bg, im_v[0, s], acc0)
                acc1 = jnp.where(jnp.logical_and(bg, keep_hi), im_v[1, s], acc1)
                acc_v[0, s] = acc0
                acc_v[1, s] = acc1
                return c2

            lax.fori_loop(0, ngroups, group_body, 0)
            pltpu.sync_copy(acc_v, out_hbm.at[b, pl.ds(ch0, 2), pl.ds(off, n)])
            return carry

        lax.fori_loop(0, total_chunks, chunk_body, 0)

    return body(tbl, frags, alphas, im4)


def kernel(fragments, alphas, ptclds, im):
    B, K, H, W = fragments.shape
    HW = H * W
    u = lax.bitcast_convert_type(
        ptclds.astype(jnp.bfloat16), jnp.uint16).astype(jnp.uint32)
    w01 = u[0] | (u[1] << 16)
    w23 = u[2] | (u[3] << 16)
    tbl = lax.bitcast_convert_type(jnp.stack([w01, w23]), jnp.int32)
    im4 = jnp.concatenate(
        [im.reshape(3, HW), jnp.zeros((1, HW), jnp.float32)], axis=0)
    out = _sc_composite(tbl, fragments.reshape(B, K, HW),
                        alphas.reshape(B, K, HW), im4)
    return out.reshape(B, 4, H, W)


# speedup vs baseline: 139.2232x; 1.6350x over previous
import functools

import jax
import jax.numpy as jnp
from jax import lax
from jax.experimental import pallas as pl
from jax.experimental.pallas import tpu as pltpu
from jax.experimental.pallas import tpu_sc as plsc

NC, NS = 2, 16
NW = NC * NS
PAIRS = NW // 2
LANES = 16


def _sc_composite(tbl, frags, alphas, im4):
    B, K, HW = frags.shape
    P = tbl.shape[1]
    ppb = HW // PAIRS
    n = min(256, ppb)
    nchunk = ppb // n
    ngroups = n // LANES
    total_chunks = B * nchunk
    half = total_chunks // 2

    mesh = plsc.VectorSubcoreMesh(core_axis_name="c", subcore_axis_name="s")

    @functools.partial(
        pl.kernel,
        out_type=jax.ShapeDtypeStruct((B, 4, HW), jnp.float32),
        mesh=mesh,
        compiler_params=pltpu.CompilerParams(needs_layout_passes=False),
        scratch_types=[
            pltpu.VMEM((P,), jnp.int32),
            pltpu.VMEM((K, n), jnp.int32),
            pltpu.VMEM((K, n), jnp.int32),
            pltpu.VMEM((K, n), jnp.float32),
            pltpu.VMEM((K, n), jnp.float32),
            pltpu.VMEM((2, n), jnp.float32),
            pltpu.VMEM((2, n), jnp.float32),
            pltpu.VMEM((2, n), jnp.float32),
            pltpu.VMEM((2, n), jnp.float32),
            pltpu.SemaphoreType.DMA,
            pltpu.SemaphoreType.DMA,
            pltpu.SemaphoreType.DMA,
            pltpu.SemaphoreType.DMA,
        ],
    )
    def body(tbl_hbm, frag_hbm, alpha_hbm, im_hbm, out_hbm,
             tbl_v, frag0, frag1, alpha0, alpha1, im0, im1, acc0_v, acc1_v,
             isem0, isem1, osem0, osem1):
        fv = (frag0, frag1)
        av = (alpha0, alpha1)
        iv = (im0, im1)
        accv = (acc0_v, acc1_v)
        isem = (isem0, isem1)
        osem = (osem0, osem1)

        wid = lax.axis_index("s") * NC + lax.axis_index("c")
        pair = wid // 2
        member = wid % 2
        ch0 = member * 2
        keep_hi = (ch0 + 1) < 3
        pltpu.sync_copy(tbl_hbm.at[member], tbl_v)

        def in_copies(i, buf):
            b = i // nchunk
            off = pair * ppb + (i % nchunk) * n
            return (
                pltpu.make_async_copy(
                    frag_hbm.at[b, :, pl.ds(off, n)], fv[buf], isem[buf]),
                pltpu.make_async_copy(
                    alpha_hbm.at[b, :, pl.ds(off, n)], av[buf], isem[buf]),
                pltpu.make_async_copy(
                    im_hbm.at[pl.ds(ch0, 2), pl.ds(off, n)], iv[buf], isem[buf]),
            )

        def out_copy(i, buf):
            b = i // nchunk
            off = pair * ppb + (i % nchunk) * n
            return pltpu.make_async_copy(
                accv[buf], out_hbm.at[b, pl.ds(ch0, 2), pl.ds(off, n)],
                osem[buf])

        for c in in_copies(0, 0):
            c.start()
        for c in in_copies(1, 1):
            c.start()

        def jbody(j, carry):
            for buf in (0, 1):
                i = 2 * j + buf
                for c in in_copies(i, buf):
                    c.wait()

                @pl.when(j > 0)
                def _():
                    out_copy(i, buf).wait()

                def group_body(g, c2):
                    s = pl.ds(g * LANES, LANES)
                    t = jnp.ones((LANES,), jnp.float32)
                    acc0 = jnp.zeros((LANES,), jnp.float32)
                    acc1 = jnp.zeros((LANES,), jnp.float32)
                    for kk in range(K):
                        idx = fv[buf][kk, s]
                        al = av[buf][kk, s]
                        a = jnp.where(idx >= 0, al, 0.0)
                        w = a * t
                        t = t * (1.0 - a)
                        safe = jnp.maximum(idx, 0)
                        gw = plsc.load_gather(tbl_v, [safe])
                        clo = plsc.bitcast(gw << 16, jnp.float32)
                        chi = plsc.bitcast(gw & jnp.int32(-65536), jnp.float32)
                        acc0 = acc0 + w * clo
                        acc1 = acc1 + w * chi
                    bg = fv[buf][0, s] < 0
                    acc0 = jnp.where(bg, iv[buf][0, s], acc0)
                    acc1 = jnp.where(jnp.logical_and(bg, keep_hi),
                                     iv[buf][1, s], acc1)
                    accv[buf][0, s] = acc0
                    accv[buf][1, s] = acc1
                    return c2

                lax.fori_loop(0, ngroups, group_body, 0)
                out_copy(i, buf).start()

                @pl.when(j < half - 1)
                def _():
                    for c in in_copies(i + 2, buf):
                        c.start()
            return carry

        lax.fori_loop(0, half, jbody, 0)
        out_copy(total_chunks - 2, 0).wait()
        out_copy(total_chunks - 1, 1).wait()

    return body(tbl, frags, alphas, im4)


def kernel(fragments, alphas, ptclds, im):
    B, K, H, W = fragments.shape
    HW = H * W
    u = lax.bitcast_convert_type(
        ptclds.astype(jnp.bfloat16), jnp.uint16).astype(jnp.uint32)
    w01 = u[0] | (u[1] << 16)
    w23 = u[2] | (u[3] << 16)
    tbl = lax.bitcast_convert_type(jnp.stack([w01, w23]), jnp.int32)
    im4 = jnp.concatenate(
        [im.reshape(3, HW), jnp.zeros((1, HW), jnp.float32)], axis=0)
    out = _sc_composite(tbl, fragments.reshape(B, K, HW),
                        alphas.reshape(B, K, HW), im4)
    return out.reshape(B, 4, H, W)


# speedup vs baseline: 146.2528x; 1.0505x over previous
import functools

import jax
import jax.numpy as jnp
from jax import lax
from jax.experimental import pallas as pl
from jax.experimental.pallas import tpu as pltpu
from jax.experimental.pallas import tpu_sc as plsc

NC, NS = 2, 16
NW = NC * NS
PAIRS = NW // 2
LANES = 16


def _sc_composite(tbl, frags, alphas, im4):
    B, K, HW = frags.shape
    P = tbl.shape[1]
    ppb = HW // PAIRS
    n = min(256, ppb)
    nchunk = ppb // n
    ngroups = n // LANES
    total_chunks = B * nchunk
    half = total_chunks // 2

    mesh = plsc.VectorSubcoreMesh(core_axis_name="c", subcore_axis_name="s")

    @functools.partial(
        pl.kernel,
        out_type=jax.ShapeDtypeStruct((B, 4, HW), jnp.float32),
        mesh=mesh,
        compiler_params=pltpu.CompilerParams(needs_layout_passes=False),
        scratch_types=[
            pltpu.VMEM((P,), jnp.int32),
            pltpu.VMEM((K, n), jnp.int32),
            pltpu.VMEM((K, n), jnp.int32),
            pltpu.VMEM((K, n), jnp.float32),
            pltpu.VMEM((K, n), jnp.float32),
            pltpu.VMEM((2, n), jnp.float32),
            pltpu.VMEM((2, n), jnp.float32),
            pltpu.VMEM((2, n), jnp.float32),
            pltpu.VMEM((2, n), jnp.float32),
            pltpu.SemaphoreType.DMA,
            pltpu.SemaphoreType.DMA,
            pltpu.SemaphoreType.DMA,
            pltpu.SemaphoreType.DMA,
        ],
    )
    def body(tbl_hbm, frag_hbm, alpha_hbm, im_hbm, out_hbm,
             tbl_v, frag0, frag1, alpha0, alpha1, im0, im1, acc0_v, acc1_v,
             isem0, isem1, osem0, osem1):
        fv = (frag0, frag1)
        av = (alpha0, alpha1)
        iv = (im0, im1)
        accv = (acc0_v, acc1_v)
        isem = (isem0, isem1)
        osem = (osem0, osem1)

        wid = lax.axis_index("s") * NC + lax.axis_index("c")
        pair = wid // 2
        member = wid % 2
        ch0 = member * 2
        keep_hi = (ch0 + 1) < 3
        pltpu.sync_copy(tbl_hbm.at[member], tbl_v)

        def in_copies(i, buf):
            b = i // nchunk
            off = pair * ppb + (i % nchunk) * n
            return (
                pltpu.make_async_copy(
                    frag_hbm.at[b, :, pl.ds(off, n)], fv[buf], isem[buf]),
                pltpu.make_async_copy(
                    alpha_hbm.at[b, :, pl.ds(off, n)], av[buf], isem[buf]),
                pltpu.make_async_copy(
                    im_hbm.at[pl.ds(ch0, 2), pl.ds(off, n)], iv[buf], isem[buf]),
            )

        def out_copy(i, buf):
            b = i // nchunk
            off = pair * ppb + (i % nchunk) * n
            return pltpu.make_async_copy(
                accv[buf], out_hbm.at[b, pl.ds(ch0, 2), pl.ds(off, n)],
                osem[buf])

        for c in in_copies(0, 0):
            c.start()
        for c in in_copies(1, 1):
            c.start()

        def jbody(j, carry):
            for buf in (0, 1):
                i = 2 * j + buf
                for c in in_copies(i, buf):
                    c.wait()

                @pl.when(j > 0)
                def _():
                    out_copy(i, buf).wait()

                @plsc.parallel_loop(0, ngroups, step=1, unroll=2)
                def group_body(g):
                    s = pl.ds(g * LANES, LANES)
                    t = jnp.ones((LANES,), jnp.float32)
                    acc0 = jnp.zeros((LANES,), jnp.float32)
                    acc1 = jnp.zeros((LANES,), jnp.float32)
                    for kk in range(K):
                        idx = fv[buf][kk, s]
                        al = av[buf][kk, s]
                        a = jnp.where(idx >= 0, al, 0.0)
                        w = a * t
                        t = t - w
                        safe = jnp.maximum(idx, 0)
                        gw = plsc.load_gather(tbl_v, [safe])
                        clo = plsc.bitcast(gw << 16, jnp.float32)
                        chi = plsc.bitcast(gw, jnp.float32)
                        acc0 = acc0 + w * clo
                        acc1 = acc1 + w * chi
                    bg = fv[buf][0, s] < 0
                    acc0 = jnp.where(bg, iv[buf][0, s], acc0)
                    acc1 = jnp.where(jnp.logical_and(bg, keep_hi),
                                     iv[buf][1, s], acc1)
                    accv[buf][0, s] = acc0
                    accv[buf][1, s] = acc1
                out_copy(i, buf).start()

                @pl.when(j < half - 1)
                def _():
                    for c in in_copies(i + 2, buf):
                        c.start()
            return carry

        lax.fori_loop(0, half, jbody, 0)
        out_copy(total_chunks - 2, 0).wait()
        out_copy(total_chunks - 1, 1).wait()

    return body(tbl, frags, alphas, im4)


def kernel(fragments, alphas, ptclds, im):
    B, K, H, W = fragments.shape
    HW = H * W
    def pack_pair(c_lo, c_hi):
        lo = lax.bitcast_convert_type(
            c_lo.astype(jnp.bfloat16), jnp.uint16).astype(jnp.uint32)
        b1 = lax.bitcast_convert_type(c_hi, jnp.uint32)
        hi = jnp.where(b1 >= jnp.uint32(0x10000),
                       (b1 + jnp.uint32(0x8000) - lo) >> 16, b1 >> 16)
        return (hi << 16) | lo

    tbl = lax.bitcast_convert_type(
        jnp.stack([pack_pair(ptclds[0], ptclds[1]),
                   pack_pair(ptclds[2], ptclds[3])]), jnp.int32)
    im4 = jnp.concatenate(
        [im.reshape(3, HW), jnp.zeros((1, HW), jnp.float32)], axis=0)
    out = _sc_composite(tbl, fragments.reshape(B, K, HW),
                        alphas.reshape(B, K, HW), im4)
    return out.reshape(B, 4, H, W)


# speedup vs baseline: 263.9147x; 1.8045x over previous
import functools

import jax
import jax.numpy as jnp
from jax import lax
from jax.experimental import pallas as pl
from jax.experimental.pallas import tpu as pltpu
from jax.experimental.pallas import tpu_sc as plsc

NC, NS = 2, 16
NW = NC * NS
PAIRS = NW // 2
LANES = 16


def _sc_composite(tbl, frags, alphas, im):
    B, K, H, W = frags.shape
    P = tbl.shape[1]
    rpp = H // PAIRS
    n = min(256, W)
    ncol = W // n
    ngroups = n // LANES
    total_chunks = B * rpp * ncol
    half = total_chunks // 2
    per_b = rpp * ncol

    mesh = plsc.VectorSubcoreMesh(core_axis_name="c", subcore_axis_name="s")

    @functools.partial(
        pl.kernel,
        out_type=jax.ShapeDtypeStruct((B, 4, H, W), jnp.float32),
        mesh=mesh,
        compiler_params=pltpu.CompilerParams(needs_layout_passes=False),
        scratch_types=[
            pltpu.VMEM((P,), jnp.int32),
            pltpu.VMEM((K, n), jnp.int32),
            pltpu.VMEM((K, n), jnp.int32),
            pltpu.VMEM((K, n), jnp.float32),
            pltpu.VMEM((K, n), jnp.float32),
            pltpu.VMEM((2, n), jnp.float32),
            pltpu.VMEM((2, n), jnp.float32),
            pltpu.VMEM((2, n), jnp.float32),
            pltpu.VMEM((2, n), jnp.float32),
            pltpu.SemaphoreType.DMA,
            pltpu.SemaphoreType.DMA,
            pltpu.SemaphoreType.DMA,
            pltpu.SemaphoreType.DMA,
        ],
    )
    def body(tbl_hbm, frag_hbm, alpha_hbm, im_hbm, out_hbm,
             tbl_v, frag0, frag1, alpha0, alpha1, im0, im1, acc0_v, acc1_v,
             isem0, isem1, osem0, osem1):
        fv = (frag0, frag1)
        av = (alpha0, alpha1)
        iv = (im0, im1)
        accv = (acc0_v, acc1_v)
        isem = (isem0, isem1)
        osem = (osem0, osem1)

        wid = lax.axis_index("s") * NC + lax.axis_index("c")
        pair = wid // 2
        member = wid % 2
        ch0 = member * 2
        keep_hi = (ch0 + 1) < 3
        pltpu.sync_copy(tbl_hbm.at[member], tbl_v)

        def pos(i):
            b = i // per_b
            rem = i % per_b
            r = pair * rpp + rem // ncol
            col = (rem % ncol) * n
            return b, r, col

        def in_copies(i, buf):
            b, r, col = pos(i)
            return (
                pltpu.make_async_copy(
                    frag_hbm.at[b, :, r, pl.ds(col, n)], fv[buf], isem[buf]),
                pltpu.make_async_copy(
                    alpha_hbm.at[b, :, r, pl.ds(col, n)], av[buf], isem[buf]),
                pltpu.make_async_copy(
                    im_hbm.at[pl.ds(member, 2), r, pl.ds(col, n)], iv[buf],
                    isem[buf]),
            )

        def out_copy(i, buf):
            b, r, col = pos(i)
            return pltpu.make_async_copy(
                accv[buf], out_hbm.at[b, pl.ds(ch0, 2), r, pl.ds(col, n)],
                osem[buf])

        for c in in_copies(0, 0):
            c.start()
        for c in in_copies(1, 1):
            c.start()

        def jbody(j, carry):
            for buf in (0, 1):
                i = 2 * j + buf
                for c in in_copies(i, buf):
                    c.wait()

                @pl.when(j > 0)
                def _():
                    out_copy(i, buf).wait()

                @plsc.parallel_loop(0, ngroups, step=1, unroll=2)
                def group_body(g):
                    s = pl.ds(g * LANES, LANES)
                    t = jnp.ones((LANES,), jnp.float32)
                    acc0 = jnp.zeros((LANES,), jnp.float32)
                    acc1 = jnp.zeros((LANES,), jnp.float32)
                    for kk in range(K):
                        idx = fv[buf][kk, s]
                        al = av[buf][kk, s]
                        a = jnp.where(idx >= 0, al, 0.0)
                        w = a * t
                        t = t - w
                        safe = jnp.maximum(idx, 0)
                        gw = plsc.load_gather(tbl_v, [safe])
                        clo = plsc.bitcast(gw << 16, jnp.float32)
                        chi = plsc.bitcast(gw, jnp.float32)
                        acc0 = acc0 + w * clo
                        acc1 = acc1 + w * chi
                    bg = fv[buf][0, s] < 0
                    im_lo = jnp.where(member == 0, iv[buf][0, s], iv[buf][1, s])
                    acc0 = jnp.where(bg, im_lo, acc0)
                    acc1 = jnp.where(jnp.logical_and(bg, keep_hi),
                                     iv[buf][1, s], acc1)
                    accv[buf][0, s] = acc0
                    accv[buf][1, s] = acc1

                out_copy(i, buf).start()

                @pl.when(j < half - 1)
                def _():
                    for c in in_copies(i + 2, buf):
                        c.start()
            return carry

        lax.fori_loop(0, half, jbody, 0)
        out_copy(total_chunks - 2, 0).wait()
        out_copy(total_chunks - 1, 1).wait()

    return body(tbl, frags, alphas, im)


def kernel(fragments, alphas, ptclds, im):
    def pack_pair(c_lo, c_hi):
        lo = lax.bitcast_convert_type(
            c_lo.astype(jnp.bfloat16), jnp.uint16).astype(jnp.uint32)
        b1 = lax.bitcast_convert_type(c_hi, jnp.uint32)
        hi = jnp.where(b1 >= jnp.uint32(0x10000),
                       (b1 + jnp.uint32(0x8000) - lo) >> 16, b1 >> 16)
        return (hi << 16) | lo

    tbl = lax.bitcast_convert_type(
        jnp.stack([pack_pair(ptclds[0], ptclds[1]),
                   pack_pair(ptclds[2], ptclds[3])]), jnp.int32)
    return _sc_composite(tbl, fragments, alphas, im)
